# Initial kernel scaffold; baseline (speedup 1.0000x reference)
import jax
import jax.numpy as jnp
from jax.experimental import pallas as pl


def kernel(x, pos, edge_attr, edge_index, batch, time, params):
    raise NotImplementedError("write your pallas kernel here")


# speedup vs baseline: 18.2918x; 18.2918x over previous
import functools

import jax
import jax.numpy as jnp
import numpy as np
from jax import lax
from jax.experimental import pallas as pl
from jax.experimental.pallas import tpu as pltpu
from jax.experimental.pallas import tpu_sc as plsc

N = 10000
NPAD = 10240
E0 = 160000
ER = 65536
EPAD = 229376
EPW = EPAD // 32
G = 16
PW = 2048
RPW = 313
NSW = 320
L = 16
BLK = 128
NB = EPW // BLK
ZROW = NPAD // 16
LDIMS = [32, 96, 160, 192, 192]
EBLK = 512
NEB = EPAD // EBLK

_mesh = plsc.VectorSubcoreMesh(core_axis_name="c", subcore_axis_name="s",
                               num_cores=2, num_subcores=16)

f32 = jnp.float32
i32 = jnp.int32


def _mm(a, b):
    return jnp.dot(a, b, precision=jax.lax.Precision.HIGHEST)


def _wid():
    return lax.axis_index("s") * 2 + lax.axis_index("c")


def _bf16_rne(x):
    u = plsc.bitcast(x, jnp.uint32)
    r = u + jnp.uint32(0x7FFF) + ((u >> jnp.uint32(16)) & jnp.uint32(1))
    return plsc.bitcast(r & jnp.uint32(0xFFFF0000), f32)


def _radius_body(px_h, py_h, pz_h, batch_h, garr_h,
                 rsrc_h, rdst_h,
                 px_v, py_v, pz_v, pxb_v, pyb_v, pzb_v, p2_v,
                 b_v, g_v, sbuf, dbuf):
    wid = _wid()
    pltpu.sync_copy(px_h, px_v)
    pltpu.sync_copy(py_h, py_v)
    pltpu.sync_copy(pz_h, pz_v)
    pltpu.sync_copy(batch_h, b_v)
    pltpu.sync_copy(garr_h, g_v)

    def p2fill(k, _):
        xf = px_v[pl.ds(k * L, L)]
        yf = py_v[pl.ds(k * L, L)]
        zf = pz_v[pl.ds(k * L, L)]
        p2_v[pl.ds(k * L, L)] = xf * xf + yf * yf + zf * zf
        pxb_v[pl.ds(k * L, L)] = _bf16_rne(xf)
        pyb_v[pl.ds(k * L, L)] = _bf16_rne(yf)
        pzb_v[pl.ds(k * L, L)] = _bf16_rne(zf)
        return 0
    lax.fori_loop(0, NPAD // L, p2fill, 0)

    def zero(k, _):
        sbuf[pl.ds(k * L, L)] = jnp.zeros((L,), i32)
        dbuf[pl.ds(k * L, L)] = jnp.zeros((L,), i32)
        return 0
    lax.fori_loop(0, PW // L, zero, 0)

    iota = lax.iota(i32, L)
    wlo = wid * RPW
    whi = jnp.minimum(wlo + RPW, N)

    def row(i, off):
        isp = jnp.full((L,), i, i32)
        xi = plsc.load_gather(pxb_v, [isp])
        yi = plsc.load_gather(pyb_v, [isp])
        zi = plsc.load_gather(pzb_v, [isp])
        p2i = plsc.load_gather(p2_v, [isp])
        bv = plsc.load_gather(b_v, [isp])
        gsv = plsc.load_gather(g_v, [bv])
        gev = plsc.load_gather(g_v, [bv + 16])
        gs = jnp.max(gsv)
        ge = jnp.max(gev)

        def chunk(c, off2):
            j0 = c * L
            jv = j0 + iota
            xj = pxb_v[pl.ds(j0, L)]
            yj = pyb_v[pl.ds(j0, L)]
            zj = pzb_v[pl.ds(j0, L)]
            p2j = p2_v[pl.ds(j0, L)]
            d2 = (p2i + p2j) - 2.0 * (xi * xj + yi * yj + zi * zj)
            m = (d2 < 25.0) & (jv != isp) & (jv >= gsv) & (jv < gev)

            def hit(o):
                mi = m.astype(i32)
                posn = o + jnp.cumsum(mi) - 1
                ok = m & (posn < PW)
                plsc.store_scatter(dbuf, [posn], jv, mask=ok)
                plsc.store_scatter(sbuf, [posn], isp, mask=ok)
                return o + jnp.sum(mi)

            return lax.cond(jnp.any(m), hit, lambda o: o, off2)

        return lax.fori_loop(gs // L, (ge + L - 1) // L, chunk, off)

    lax.fori_loop(wlo, whi, row, 0)
    pltpu.sync_copy(sbuf, rsrc_h.at[pl.ds(wid * PW, PW)])
    pltpu.sync_copy(dbuf, rdst_h.at[pl.ds(wid * PW, PW)])


_radius_call = pl.kernel(
    _radius_body,
    out_type=[jax.ShapeDtypeStruct((ER,), i32),
              jax.ShapeDtypeStruct((ER,), i32)],
    mesh=_mesh,
    scratch_types=[pltpu.VMEM((NPAD,), f32), pltpu.VMEM((NPAD,), f32),
                   pltpu.VMEM((NPAD,), f32), pltpu.VMEM((NPAD,), f32),
                   pltpu.VMEM((NPAD,), f32), pltpu.VMEM((NPAD,), f32),
                   pltpu.VMEM((NPAD,), f32), pltpu.VMEM((NPAD,), i32),
                   pltpu.VMEM((32,), i32),
                   pltpu.VMEM((PW,), i32), pltpu.VMEM((PW,), i32)],
    compiler_params=pltpu.CompilerParams(needs_layout_passes=False, use_tc_tiling_on_sc=False),
)


def _prep_body(src_h, dst_h, px_h, py_h, pz_h, batch_h, te_h,
               geo_h, esig_h, ns_h,
               px_v, py_v, pz_v, b_v, sblk, dblk, bs_v, geo_v, esig_v, sem):
    wid = _wid()
    pltpu.sync_copy(px_h, px_v)
    pltpu.sync_copy(py_h, py_v)
    pltpu.sync_copy(pz_h, pz_v)
    pltpu.sync_copy(batch_h, b_v)
    iota = lax.iota(i32, L)

    def block(b, _):
        gb = wid * EPW + b * BLK
        pltpu.sync_copy(src_h.at[pl.ds(gb, BLK)], sblk)
        pltpu.sync_copy(dst_h.at[pl.ds(gb, BLK)], dblk)

        def chunk(k, _2):
            lo = k * L
            s16 = sblk[pl.ds(lo, L)]
            d16 = dblk[pl.ds(lo, L)]
            xs = plsc.load_gather(px_v, [s16])
            ys = plsc.load_gather(py_v, [s16])
            zs = plsc.load_gather(pz_v, [s16])
            xd = plsc.load_gather(px_v, [d16])
            yd = plsc.load_gather(py_v, [d16])
            zd = plsc.load_gather(pz_v, [d16])
            dx = xd - xs
            dy = yd - ys
            dz = zd - zs
            eid = gb + lo + iota
            w = jnp.where(eid < E0, 1.0,
                          jnp.where(s16 != d16, 1.0, 0.0)).astype(f32)
            le = lo + iota
            plsc.store_scatter(geo_v, [le * 4], dx)
            plsc.store_scatter(geo_v, [le * 4 + 1], dy)
            plsc.store_scatter(geo_v, [le * 4 + 2], dz)
            plsc.store_scatter(geo_v, [le * 4 + 3], w)
            bv = plsc.load_gather(b_v, [s16])
            bs_v[pl.ds(lo, L)] = bv
            return 0

        lax.fori_loop(0, BLK // L, chunk, 0)
        pltpu.async_copy(te_h.at[bs_v], esig_v, sem).wait()
        pltpu.sync_copy(esig_v, esig_h.at[pl.ds(gb, BLK)])
        pltpu.sync_copy(geo_v, geo_h.at[pl.ds(gb * 4, BLK * 4)])
        return 0

    lax.fori_loop(0, NB, block, 0)

    base = wid * NSW
    for off, cnt in ((0, 128), (128, 128), (256, 64)):
        def chunk2(k, _2, off=off):
            idx = jnp.minimum(base + off + k * L + iota, N - 1)
            bs_v[pl.ds(k * L, L)] = plsc.load_gather(b_v, [idx])
            return 0
        lax.fori_loop(0, BLK // L, chunk2, 0)
        pltpu.async_copy(te_h.at[bs_v], esig_v, sem).wait()
        pltpu.sync_copy(esig_v.at[pl.ds(0, cnt)],
                        ns_h.at[pl.ds(base + off, cnt)])


_prep_call = pl.kernel(
    _prep_body,
    out_type=[jax.ShapeDtypeStruct((EPAD * 4,), f32),
              jax.ShapeDtypeStruct((EPAD, 32), f32),
              jax.ShapeDtypeStruct((NPAD, 32), f32)],
    mesh=_mesh,
    scratch_types=[pltpu.VMEM((NPAD,), f32), pltpu.VMEM((NPAD,), f32),
                   pltpu.VMEM((NPAD,), f32), pltpu.VMEM((NPAD,), i32),
                   pltpu.VMEM((BLK,), i32), pltpu.VMEM((BLK,), i32),
                   pltpu.VMEM((BLK,), i32),
                   pltpu.VMEM((BLK * 4,), f32), pltpu.VMEM((BLK, 32), f32),
                   pltpu.SemaphoreType.DMA],
    compiler_params=pltpu.CompilerParams(needs_layout_passes=False, use_tc_tiling_on_sc=False),
)


def _gather_body(src_h, dst_h, h_h, h32_h,
                 hsrc_h, hd32_h,
                 sblk, dblk, hbuf, h32buf, sem, sem2):
    wid = _wid()

    def block(b, _):
        gb = wid * EPW + b * BLK
        pltpu.sync_copy(src_h.at[pl.ds(gb, BLK)], sblk)
        pltpu.sync_copy(dst_h.at[pl.ds(gb, BLK)], dblk)
        d1 = pltpu.async_copy(h_h.at[sblk], hbuf, sem)
        d2 = pltpu.async_copy(h32_h.at[dblk], h32buf, sem2)
        d1.wait()
        d2.wait()
        pltpu.sync_copy(hbuf, hsrc_h.at[pl.ds(gb, BLK)])
        pltpu.sync_copy(h32buf, hd32_h.at[pl.ds(gb, BLK)])
        return 0

    lax.fori_loop(0, NB, block, 0)


@functools.cache
def _make_gather(din):
    return pl.kernel(
        _gather_body,
        out_type=[jax.ShapeDtypeStruct((EPAD, din), f32),
                  jax.ShapeDtypeStruct((EPAD, 32), f32)],
        mesh=_mesh,
        scratch_types=[pltpu.VMEM((BLK,), i32), pltpu.VMEM((BLK,), i32),
                       pltpu.VMEM((BLK, din), f32), pltpu.VMEM((BLK, 32), f32),
                       pltpu.SemaphoreType.DMA, pltpu.SemaphoreType.DMA],
        compiler_params=pltpu.CompilerParams(needs_layout_passes=False, use_tc_tiling_on_sc=False),
    )


def _scatter_body_cnt(msg_h, dst_h, geo_h, zin_h, zc_h,
                      pa_h, pb_h, ca_h, cb_h,
                      shared, sharedc, dblk, mbuf, gbuf, wbuf):
    sid = lax.axis_index("s")
    cid = lax.axis_index("c")
    wid = _wid()
    iota = lax.iota(i32, L)
    r0 = sid * ZROW
    pltpu.sync_copy(zin_h.at[pl.ds(r0, ZROW)], shared.at[pl.ds(r0, ZROW)])
    pltpu.sync_copy(zc_h.at[pl.ds(r0, ZROW)], sharedc.at[pl.ds(r0, ZROW)])
    pltpu.sync_copy(zc_h.at[pl.ds(0, BLK)], wbuf)
    plsc.subcore_barrier()

    def block(b, _):
        gb = wid * EPW + b * BLK
        pltpu.sync_copy(dst_h.at[pl.ds(gb, BLK)], dblk)
        pltpu.sync_copy(msg_h.at[pl.ds(gb, BLK)], mbuf)
        pltpu.sync_copy(mbuf, shared.at[dblk], add=True)
        pltpu.sync_copy(geo_h.at[pl.ds(gb * 4, BLK * 4)], gbuf)

        def wch(k, _2):
            le = k * L + iota
            wv = plsc.load_gather(gbuf, [le * 4 + 3])
            plsc.store_scatter(wbuf, [le, jnp.zeros((L,), i32)], wv)
            return 0

        lax.fori_loop(0, BLK // L, wch, 0)
        pltpu.sync_copy(wbuf, sharedc.at[dblk], add=True)
        return 0

    lax.fori_loop(0, NB, block, 0)
    plsc.subcore_barrier()

    @pl.when(cid == 0)
    def _():
        pltpu.sync_copy(shared.at[pl.ds(r0, ZROW)], pa_h.at[pl.ds(r0, ZROW)])
        pltpu.sync_copy(sharedc.at[pl.ds(r0, ZROW)], ca_h.at[pl.ds(r0, ZROW)])

    @pl.when(cid == 1)
    def _():
        pltpu.sync_copy(shared.at[pl.ds(r0, ZROW)], pb_h.at[pl.ds(r0, ZROW)])
        pltpu.sync_copy(sharedc.at[pl.ds(r0, ZROW)], cb_h.at[pl.ds(r0, ZROW)])


def _scatter_body(msg_h, dst_h, zin_h,
                  pa_h, pb_h,
                  shared, dblk, mbuf, *, sblk):
    sid = lax.axis_index("s")
    cid = lax.axis_index("c")
    wid = _wid()
    r0 = sid * ZROW
    pltpu.sync_copy(zin_h.at[pl.ds(r0, ZROW)], shared.at[pl.ds(r0, ZROW)])
    plsc.subcore_barrier()

    def block(b, _):
        gb = wid * EPW + b * sblk
        pltpu.sync_copy(dst_h.at[pl.ds(gb, sblk)], dblk)
        pltpu.sync_copy(msg_h.at[pl.ds(gb, sblk)], mbuf)
        pltpu.sync_copy(mbuf, shared.at[dblk], add=True)
        return 0

    lax.fori_loop(0, EPW // sblk, block, 0)
    plsc.subcore_barrier()

    @pl.when(cid == 0)
    def _():
        pltpu.sync_copy(shared.at[pl.ds(r0, ZROW)], pa_h.at[pl.ds(r0, ZROW)])

    @pl.when(cid == 1)
    def _():
        pltpu.sync_copy(shared.at[pl.ds(r0, ZROW)], pb_h.at[pl.ds(r0, ZROW)])


@functools.cache
def _make_scatter(din, with_cnt):
    if with_cnt:
        return pl.kernel(
            _scatter_body_cnt,
            out_type=[jax.ShapeDtypeStruct((NPAD, din), f32),
                      jax.ShapeDtypeStruct((NPAD, din), f32),
                      jax.ShapeDtypeStruct((NPAD, 16), f32),
                      jax.ShapeDtypeStruct((NPAD, 16), f32)],
            mesh=_mesh,
            scratch_types=[pltpu.VMEM_SHARED((NPAD, din), f32),
                           pltpu.VMEM_SHARED((NPAD, 16), f32),
                           pltpu.VMEM((BLK,), i32),
                           pltpu.VMEM((BLK, din), f32),
                           pltpu.VMEM((BLK * 4,), f32),
                           pltpu.VMEM((BLK, 16), f32)],
            compiler_params=pltpu.CompilerParams(needs_layout_passes=False, use_tc_tiling_on_sc=False),
        )
    sblk = {160: 64, 192: 32}.get(din, BLK)
    return pl.kernel(
        functools.partial(_scatter_body, sblk=sblk),
        out_type=[jax.ShapeDtypeStruct((NPAD, din), f32),
                  jax.ShapeDtypeStruct((NPAD, din), f32)],
        mesh=_mesh,
        scratch_types=[pltpu.VMEM_SHARED((NPAD, din), f32),
                       pltpu.VMEM((sblk,), i32),
                       pltpu.VMEM((sblk, din), f32)],
        compiler_params=pltpu.CompilerParams(needs_layout_passes=False, use_tc_tiling_on_sc=False),
    )


_SQ3 = float(np.sqrt(3.0))
_C2 = float(np.sqrt(15.0))
_C5H = float(np.sqrt(5.0) / 2.0)
_OFFS = np.linspace(0.0, 5.0, 50).astype(np.float32)
_COEFF = float(-0.5 / (_OFFS[1] - _OFFS[0]) ** 2)


def _edge_mlp_body(ea_ref, es_ref, geo_ref, w1a, w1b, w1c, b1, w2, b2,
                   e_ref, sh_ref):
    geo = geo_ref[...]
    dx = geo[:, 0:1]
    dy = geo[:, 1:2]
    dz = geo[:, 2:3]
    d = jnp.sqrt(dx * dx + dy * dy + dz * dz)
    offs = (lax.broadcasted_iota(i32, (1, 50), 1).astype(f32)
            * np.float32(5.0 / 49.0))
    sm = jnp.exp(_COEFF * (d - offs) ** 2)
    e1 = (_mm(ea_ref[...], w1a[...]) + _mm(es_ref[...], w1b[...])
          + _mm(sm, w1c[...]) + b1[...])
    e_ref[...] = _mm(jnp.maximum(e1, 0.0), w2[...]) + b2[...]
    dn = jnp.maximum(d, 1e-9)
    vx = dx / dn
    vy = dy / dn
    vz = dz / dn
    one = jnp.ones_like(vx)
    zero = jnp.zeros_like(vx)
    sh_ref[...] = jnp.concatenate(
        [one, _SQ3 * vx, _SQ3 * vy, _SQ3 * vz,
         _C2 * vx * vy, _C2 * vy * vz, _C5H * (3.0 * vz * vz - 1.0),
         _C2 * vx * vz, (_C2 / 2.0) * (vx * vx - vy * vy),
         zero, zero, zero], axis=1)


def _edge_mlp_call(eaf, esig, geo, w1, b1, w2, b2):
    gspec = lambda r, c: pl.BlockSpec((EBLK, c), lambda i: (i, 0))
    wspec = lambda a: pl.BlockSpec(a.shape, lambda i: (0, 0))
    w1a, w1b, w1c = w1[0:4], w1[4:36], w1[36:86]
    b1r, b2r = b1.reshape(1, -1), b2.reshape(1, -1)
    return pl.pallas_call(
        _edge_mlp_body,
        grid=(NEB,),
        in_specs=[gspec(EBLK, 4), gspec(EBLK, 32), gspec(EBLK, 4),
                  wspec(w1a), wspec(w1b), wspec(w1c), wspec(b1r),
                  wspec(w2), wspec(b2r)],
        out_specs=[pl.BlockSpec((EBLK, 32), lambda i: (i, 0)),
                   pl.BlockSpec((EBLK, 12), lambda i: (i, 0))],
        out_shape=[jax.ShapeDtypeStruct((EPAD, 32), f32),
                   jax.ShapeDtypeStruct((EPAD, 12), f32)],
    )(eaf, esig, geo, w1a, w1b, w1c, b1r, w2, b2r)


def _node_mlp_body(x_ref, ns_ref, w1a, w1b, b1, w2, b2, h_ref):
    h1 = _mm(x_ref[...], w1a[...]) + _mm(ns_ref[...], w1b[...]) + b1[...]
    h_ref[...] = _mm(jnp.maximum(h1, 0.0), w2[...]) + b2[...]


def _node_mlp_call(x, ns, w1, b1, w2, b2):
    w1a, w1b = w1[0:74], w1[74:106]
    b1r, b2r = b1.reshape(1, -1), b2.reshape(1, -1)
    wspec = lambda a: pl.BlockSpec(a.shape, lambda i: (0, 0))
    return pl.pallas_call(
        _node_mlp_body,
        grid=(25,),
        in_specs=[pl.BlockSpec((400, 74), lambda i: (i, 0)),
                  pl.BlockSpec((400, 32), lambda i: (i, 0)),
                  wspec(w1a), wspec(w1b), wspec(b1r), wspec(w2), wspec(b2r)],
        out_specs=pl.BlockSpec((400, 32), lambda i: (i, 0)),
        out_shape=jax.ShapeDtypeStruct((N, 32), f32),
    )(x, ns, w1a, w1b, b1r, w2, b2r)


def _layer_mlp_body(e_ref, hs_ref, hd_ref, sh_ref, geo_ref,
                    wa1, wa2, wa3, ba, wb, bb, u, msg_ref):
    hs = hs_ref[...]
    g1 = (_mm(e_ref[...], wa1[...]) + _mm(hs[:, :32], wa2[...])
          + _mm(hd_ref[...], wa3[...]) + ba[...])
    gate = _mm(jnp.maximum(g1, 0.0), wb[...]) + bb[...]
    s = _mm(sh_ref[...], u[...])
    msg_ref[...] = hs * gate * s * geo_ref[...][:, 3:4]


def _layer_mlp_call(e, hsrc, hd32, sh, geo, lp, din):
    wa = lp['Wa']
    wa1, wa2, wa3 = wa[0:32], wa[32:64], wa[64:96]
    ba = lp['ba'].reshape(1, -1)
    bb = lp['bb'].reshape(1, -1)
    u12 = jnp.pad(lp['u'], ((0, 3), (0, 0)))
    wspec = lambda a: pl.BlockSpec(a.shape, lambda i: (0, 0))
    return pl.pallas_call(
        _layer_mlp_body,
        grid=(NEB,),
        in_specs=[pl.BlockSpec((EBLK, 32), lambda i: (i, 0)),
                  pl.BlockSpec((EBLK, din), lambda i: (i, 0)),
                  pl.BlockSpec((EBLK, 32), lambda i: (i, 0)),
                  pl.BlockSpec((EBLK, 12), lambda i: (i, 0)),
                  pl.BlockSpec((EBLK, 4), lambda i: (i, 0)),
                  wspec(wa1), wspec(wa2), wspec(wa3), wspec(ba),
                  wspec(lp['Wb']), wspec(bb), wspec(u12)],
        out_specs=pl.BlockSpec((EBLK, din), lambda i: (i, 0)),
        out_shape=jax.ShapeDtypeStruct((EPAD, din), f32),
    )(e, hsrc, hd32, sh, geo, wa1, wa2, wa3, ba, lp['Wb'], bb, u12)


NBK = 32
NBR = NPAD // NBK


def _agg_body(pa, pb, ca, cb, h_ref, wm, bm, out_ref, sum_ref, sq_ref,
              *, din, dout):
    i = pl.program_id(0)
    valid = (lax.broadcasted_iota(i32, (NBR, 1), 0) + i * NBR) < N
    cnt = ca[...][:, 0:1] + cb[...][:, 0:1]
    agg = (pa[...] + pb[...]) / jnp.maximum(cnt, 1.0)
    out = _mm(agg, wm[...]) + bm[...]
    h = h_ref[...]
    if dout > din:
        h = jnp.concatenate([h, jnp.zeros((NBR, dout - din), f32)], axis=1)
    out = jnp.where(valid, out + h, 0.0)
    out_ref[...] = out
    sum_ref[...] = jnp.sum(out, axis=0, keepdims=True).reshape(1, 1, -1)
    sq_ref[...] = jnp.sum(out * out, axis=0, keepdims=True).reshape(1, 1, -1)


def _agg_call(pa, pb, ca, cb, h, lp, din, dout):
    bm = lp['bm'].reshape(1, -1)
    wspec = lambda a: pl.BlockSpec(a.shape, lambda i: (0, 0))
    return pl.pallas_call(
        functools.partial(_agg_body, din=din, dout=dout),
        grid=(NBK,),
        in_specs=[pl.BlockSpec((NBR, din), lambda i: (i, 0)),
                  pl.BlockSpec((NBR, din), lambda i: (i, 0)),
                  pl.BlockSpec((NBR, 16), lambda i: (i, 0)),
                  pl.BlockSpec((NBR, 16), lambda i: (i, 0)),
                  pl.BlockSpec((NBR, din), lambda i: (i, 0)),
                  wspec(lp['Wm']), wspec(bm)],
        out_specs=[pl.BlockSpec((NBR, dout), lambda i: (i, 0)),
                   pl.BlockSpec((1, 1, dout), lambda i: (i, 0, 0)),
                   pl.BlockSpec((1, 1, dout), lambda i: (i, 0, 0))],
        out_shape=[jax.ShapeDtypeStruct((NPAD, dout), f32),
                   jax.ShapeDtypeStruct((NBK, 1, dout), f32),
                   jax.ShapeDtypeStruct((NBK, 1, dout), f32)],
    )(pa, pb, ca, cb, h, lp['Wm'], bm)


def _norm_body(out_ref, sums, sqs, gamma, beta, hn_ref, h32_ref):
    mu = jnp.sum(sums[...], axis=0) / N
    var = jnp.sum(sqs[...], axis=0) / N - mu * mu
    hn = (gamma[...] * (out_ref[...] - mu) / jnp.sqrt(var + 1e-5)
          + beta[...])
    hn_ref[...] = hn
    h32_ref[...] = hn[:, :32]


def _fin_body(out_ref, sums, sqs, gamma, beta, f1, fb1, f2, fb2, y_ref):
    mu = jnp.sum(sums[...], axis=0) / N
    var = jnp.sum(sqs[...], axis=0) / N - mu * mu
    hn = (gamma[...] * (out_ref[...] - mu) / jnp.sqrt(var + 1e-5)
          + beta[...])
    g = jnp.tanh(_mm(hn, f1[...]) + fb1[...])
    y_ref[...] = _mm(g, f2[...]) + fb2[...]


def _norm_call(out, sums, sqs, lp, dout, fin=None):
    gamma = lp['gamma'].reshape(1, -1)
    beta = lp['beta'].reshape(1, -1)
    wspec = lambda a: pl.BlockSpec(a.shape, lambda i: (0, 0))
    sspec = pl.BlockSpec((NBK, 1, dout), lambda i: (0, 0, 0))
    base_specs = [pl.BlockSpec((NBR, dout), lambda i: (i, 0)),
                  sspec, sspec, wspec(gamma), wspec(beta)]
    if fin is None:
        return pl.pallas_call(
            _norm_body,
            grid=(NBK,),
            in_specs=base_specs,
            out_specs=[pl.BlockSpec((NBR, dout), lambda i: (i, 0)),
                       pl.BlockSpec((NBR, 32), lambda i: (i, 0))],
            out_shape=[jax.ShapeDtypeStruct((NPAD, dout), f32),
                       jax.ShapeDtypeStruct((NPAD, 32), f32)],
        )(out, sums, sqs, gamma, beta)
    f1, fb1, f2, fb2 = fin
    fb1 = fb1.reshape(1, -1)
    fb2 = fb2.reshape(1, -1)
    return pl.pallas_call(
        _fin_body,
        grid=(NBK,),
        in_specs=base_specs + [wspec(f1), wspec(fb1), wspec(f2), wspec(fb2)],
        out_specs=pl.BlockSpec((NBR, 3), lambda i: (i, 0)),
        out_shape=jax.ShapeDtypeStruct((NPAD, 3), f32),
    )(out, sums, sqs, gamma, beta, f1, fb1, f2, fb2)


def kernel(x, pos, edge_attr, edge_index, batch, time, params):
    px = jnp.pad(pos[:, 0], (0, NPAD - N), constant_values=1e9)
    py = jnp.pad(pos[:, 1], (0, NPAD - N), constant_values=1e9)
    pz = jnp.pad(pos[:, 2], (0, NPAD - N), constant_values=1e9)
    batch_p = jnp.pad(batch, (0, NPAD - N))
    starts = jnp.searchsorted(batch, jnp.arange(G, dtype=i32)).astype(i32)
    ends = jnp.searchsorted(batch, jnp.arange(1, G + 1, dtype=i32)).astype(i32)
    garr = jnp.concatenate([starts, ends])

    half = 16
    freqs = jnp.exp(-jnp.log(10000.0) * jnp.arange(half) / half)
    targs = time.astype(f32)[:, None] * freqs[None, :]
    te = jnp.concatenate([jnp.sin(targs), jnp.cos(targs)], axis=-1)

    r_src, r_dst = _radius_call(px, py, pz, batch_p, garr)
    zpad = jnp.zeros((EPAD - E0 - ER,), i32)
    src_f = jnp.concatenate([edge_index[0], r_src, zpad])
    dst_f = jnp.concatenate([edge_index[1], r_dst, zpad])

    geo_flat, esig, ns = _prep_call(src_f, dst_f, px, py, pz, batch_p, te)
    geo = geo_flat.reshape(EPAD, 4)
    eaf = jnp.concatenate(
        [edge_attr, jnp.zeros((EPAD - E0, edge_attr.shape[1]), f32)], axis=0)

    e, sh = _edge_mlp_call(eaf, esig, geo, params['edge_W1'],
                           params['edge_b1'], params['edge_W2'],
                           params['edge_b2'])
    h = _node_mlp_call(x, ns[:N], params['node_W1'], params['node_b1'],
                       params['node_W2'], params['node_b2'])
    h = jnp.pad(h, ((0, NPAD - N), (0, 0)))
    h32 = h

    ca = cb = None
    for l in range(4):
        lp = params['layers'][l]
        din, dout = LDIMS[l], LDIMS[l + 1]
        hsrc, hd32 = _make_gather(din)(src_f, dst_f, h, h32)
        msg = _layer_mlp_call(e, hsrc, hd32, sh, geo, lp, din)
        zin = jnp.zeros((NPAD, din), f32)
        if l == 0:
            zc = jnp.zeros((NPAD, 16), f32)
            pa, pb, ca, cb = _make_scatter(din, True)(
                msg, dst_f, geo_flat, zin, zc)
        else:
            pa, pb = _make_scatter(din, False)(msg, dst_f, zin)
        out, sums, sqs = _agg_call(pa, pb, ca, cb, h, lp, din, dout)
        if l < 3:
            h, h32 = _norm_call(out, sums, sqs, lp, dout)
        else:
            y = _norm_call(out, sums, sqs, lp, dout,
                           fin=(params['fin_W1'], params['fin_b1'],
                                params['fin_W2'], params['fin_b2']))
    return y[:N]
